# baseline (speedup 1.0000x reference)
import functools

import jax
import jax.numpy as jnp
from jax import lax
from jax.experimental import pallas as pl
from jax.experimental.pallas import tpu as pltpu
from jax.experimental.pallas import tpu_sc as plsc

EMBED = 64
HALF = EMBED // 2

_NC = 2
_NS = 16
_NW = _NC * _NS
_CHUNK = 625


def _table_body(ts_ref, out_ref):
    t = ts_ref[:, :]
    col = lax.broadcasted_iota(jnp.int32, out_ref.shape, 1)
    is_sin = col < HALF
    k = jnp.where(is_sin, col, col - HALF).astype(jnp.float32)
    inv_freq = jnp.exp(k * (-2.0 * jnp.log(10000.0) / EMBED))
    phase = t * inv_freq
    out_ref[:, :] = jnp.where(is_sin, jnp.sin(phase), jnp.cos(phase))


def _build_table(time_step):
    g = time_step.shape[0]
    return pl.pallas_call(
        _table_body,
        out_shape=jax.ShapeDtypeStruct((g, EMBED), jnp.float32),
    )(time_step.reshape(g, 1))


def _gather_body(n_chunks, edge_hbm, batch_hbm, table_hbm, out_hbm,
                 idx_v, g_v, rows_v, sem_e, sem_b, sem_t, sem_w):
    wid = lax.axis_index("s") * _NC + lax.axis_index("c")
    chunk0 = wid * n_chunks

    h_e = [None] * n_chunks
    h_b = [None] * n_chunks
    h_t = [None] * n_chunks
    h_w = [None] * n_chunks

    def issue_e(i):
        h_e[i] = pltpu.async_copy(edge_hbm.at[chunk0 + i], idx_v[i % 3],
                                  sem_e[i % 3])

    def issue_b(i):
        h_b[i] = pltpu.async_copy(batch_hbm.at[idx_v[i % 3]], g_v[i % 2],
                                  sem_b[i % 2])

    def issue_t(i):
        h_t[i] = pltpu.async_copy(table_hbm.at[g_v[i % 2]], rows_v[i % 2],
                                  sem_t[i % 2])

    def issue_w(i):
        h_w[i] = pltpu.async_copy(
            rows_v[i % 2],
            out_hbm.at[pl.ds((chunk0 + i) * _CHUNK, _CHUNK)],
            sem_w[i % 2])

    issue_e(0)
    issue_e(1)
    for i in range(n_chunks):
        if i + 2 < n_chunks:
            issue_e(i + 2)
        if i == 0:
            h_e[0].wait()
            issue_b(0)
        if i >= 1:
            h_t[i - 1].wait()
            issue_w(i - 1)
        if i + 1 < n_chunks:
            h_e[i + 1].wait()
            issue_b(i + 1)
        h_b[i].wait()
        if i >= 2:
            h_w[i - 2].wait()
        issue_t(i)
    h_t[n_chunks - 1].wait()
    issue_w(n_chunks - 1)
    h_w[n_chunks - 2].wait()
    h_w[n_chunks - 1].wait()


def _sc_gather(edge_row, batch, table):
    e = edge_row.shape[0]
    n_chunks = e // (_NW * _CHUNK)
    edge2d = edge_row.reshape(e // _CHUNK, _CHUNK)
    mesh = plsc.VectorSubcoreMesh(core_axis_name="c", subcore_axis_name="s")
    run = pl.kernel(
        functools.partial(_gather_body, n_chunks),
        out_type=jax.ShapeDtypeStruct((e, EMBED), jnp.float32),
        mesh=mesh,
        scratch_types=[
            [pltpu.VMEM((_CHUNK,), jnp.int32) for _ in range(3)],
            [pltpu.VMEM((_CHUNK,), jnp.int32) for _ in range(2)],
            [pltpu.VMEM((_CHUNK, EMBED), jnp.float32) for _ in range(2)],
            [pltpu.SemaphoreType.DMA for _ in range(3)],
            [pltpu.SemaphoreType.DMA for _ in range(2)],
            [pltpu.SemaphoreType.DMA for _ in range(2)],
            [pltpu.SemaphoreType.DMA for _ in range(2)],
        ],
        compiler_params=pltpu.CompilerParams(use_tc_tiling_on_sc=False),
    )
    return run(edge2d, batch, table)


def kernel(time_step, batch, edge_index):
    table = _build_table(time_step)
    edge_row = edge_index[0]
    e = edge_row.shape[0]
    tile = _NW * _CHUNK
    pad = (-e) % tile
    if pad:
        edge_row = jnp.concatenate(
            [edge_row, jnp.zeros((pad,), dtype=edge_row.dtype)])
    out = _sc_gather(edge_row, batch, table)
    if pad:
        out = out[:e]
    return out
